# baseline (speedup 1.0000x reference)
import functools

import jax
import jax.numpy as jnp
from jax import lax
from jax.experimental import pallas as pl
from jax.experimental.pallas import tpu as pltpu
from jax.experimental.pallas import tpu_sc as plsc

N_NODES = 10000
N_EDGES = 320000
D_IN = 128
D_HID = 256
D_OUT = 64

NC = 2
NS = 16
CH = 64
NCH = 314
EPAD = NS * NCH * CH
NB = 3
AH = NB - 1
NQ = 8
IA = 5
Dh = 128
G = Dh // 16


def _make_sc_segment_sum(table_rows, acc_rows, dr, mode):
  mesh = plsc.VectorSubcoreMesh(core_axis_name="c", subcore_axis_name="s")

  @functools.partial(
      pl.kernel,
      out_type=jax.ShapeDtypeStruct((NC, acc_rows, Dh), jnp.float32),
      mesh=mesh,
      scratch_types=[
          pltpu.VMEM((NQ, CH), jnp.int32),
          pltpu.VMEM((NQ, CH), jnp.int32),
          pltpu.VMEM((NQ, CH), jnp.float32),
          pltpu.VMEM((NB, CH, Dh), jnp.float32),
          pltpu.VMEM((2, CH, Dh), jnp.float32),
          pltpu.VMEM_SHARED((acc_rows, Dh), jnp.float32),
          pltpu.SemaphoreType.DMA((NB,)),
          pltpu.SemaphoreType.DMA((2,)),
          pltpu.SemaphoreType.DMA((8,)),
      ],
  )
  def sc_kernel(x_hbm, src_hbm, dst_hbm, w_hbm, out_hbm,
                src_v, dst_v, w_v, rows_v, msg_v, acc_sh,
                gsem, ssem, isem):
    c = lax.axis_index("c")
    s = lax.axis_index("s")

    def idx_descs(j):
      q = lax.rem(j, NQ)
      sem = isem.at[lax.rem(j, 8)]
      return ((src_hbm.at[s, j], src_v.at[q], sem),
              (dst_hbm.at[s, j], dst_v.at[q], sem),
              (w_hbm.at[s, j], w_v.at[q], sem))

    def idx_transform(j):
      q = lax.rem(j, NQ)
      if mode == "node":
        lo = c * acc_rows
        for e0 in range(0, CH, 16):
          sl = pl.ds(e0, 16)
          d16 = dst_v[q, sl]
          ok = (d16 >= lo) & (d16 < lo + acc_rows)
          dst_v[q, sl] = jnp.where(ok, d16 - lo, 0)
          w_v[q, sl] = jnp.where(ok, w_v[q, sl], 0.0)
      else:
        off = c * N_NODES
        for e0 in range(0, CH, 16):
          sl = pl.ds(e0, 16)
          src_v[q, sl] = src_v[q, sl] + off

    def idx_start(j):
      for a, v, sem in idx_descs(j):
        pltpu.make_async_copy(a, v, sem).start()

    def idx_wait(j):
      for a, v, sem in idx_descs(j):
        pltpu.make_async_copy(a, v, sem).wait()

    def gather_start(b, k):
      pltpu.make_async_copy(
          x_hbm.at[src_v.at[lax.rem(k, NQ)]], rows_v.at[b],
          gsem.at[b]).start()

    def gather_wait(b, k):
      pltpu.make_async_copy(
          x_hbm.at[src_v.at[lax.rem(k, NQ)]], rows_v.at[b],
          gsem.at[b]).wait()

    def scatter_start(k):
      pltpu.make_async_copy(
          msg_v.at[lax.rem(k, 2)], acc_sh.at[dst_v.at[lax.rem(k, NQ)]],
          ssem.at[lax.rem(k, 2)]).start(add=True)

    def scatter_wait(k):
      pltpu.make_async_copy(
          msg_v.at[lax.rem(k, 2)], acc_sh.at[dst_v.at[lax.rem(k, NQ)]],
          ssem.at[lax.rem(k, 2)]).wait()

    def scale(b, k):
      q = lax.rem(k, NQ)
      mb = lax.rem(k, 2)
      for e0 in range(0, CH, 16):
        w16 = w_v[q, pl.ds(e0, 16)]
        for j in range(16):
          ws = jnp.full((16,), w16[j], jnp.float32)
          e = e0 + j
          for g in range(G):
            sl = pl.ds(16 * g, 16)
            msg_v[mb, e, sl] = rows_v[b, e, sl] * ws

    for j in range(AH):
      for a, v, _ in idx_descs(j):
        pltpu.sync_copy(a, v)
      idx_transform(jnp.int32(j))
    for j in range(AH, IA):
      idx_start(j)
    for j in range(AH):
      gather_start(j, j)

    zeros16 = jnp.zeros((16,), jnp.float32)

    def zero_row(r, carry):
      for g in range(G):
        msg_v[1, r, 16 * g:16 * (g + 1)] = zeros16
      return carry

    lax.fori_loop(0, CH, zero_row, 0)
    base = pl.multiple_of(jnp.minimum(dr * s, acc_rows - dr), 8)
    for j in range(dr // CH):
      pltpu.sync_copy(msg_v.at[1], acc_sh.at[pl.ds(base + j * CH, CH)])
    plsc.subcore_barrier()

    def step(k, b):
      gather_wait(b, k)
      @pl.when(k >= 2)
      def _wait_prev():
        scatter_wait(k - 2)

      @pl.when(k + AH < NCH)
      def _next_gather():
        idx_wait(k + AH)
        idx_transform(k + AH)
        gather_start((b + AH) % NB, k + AH)

      scale(b, k)
      scatter_start(k)

      @pl.when(k + IA < NCH)
      def _next_idx():
        idx_start(k + IA)

    def outer(ko, carry):
      for b in range(NB):
        step(NB * ko + b, b)
      return carry

    lax.fori_loop(0, NCH // NB, outer, 0)

    for k in range(NCH - NCH % NB, NCH):
      step(jnp.int32(k), k % NB)
    for k in range(NCH - 2, NCH):
      scatter_wait(k)

    plsc.subcore_barrier()

    pltpu.sync_copy(acc_sh.at[pl.ds(base, dr)],
                    out_hbm.at[c, pl.ds(base, dr)])

  return sc_kernel


_sc_seg_l1 = _make_sc_segment_sum(N_NODES, N_NODES // 2, 320, "node")
_sc_seg_l2 = _make_sc_segment_sum(NC * N_NODES, N_NODES, 640, "col")


def _mm(a, w):
  return lax.dot_general(a, w, (((1,), (0,)), ((), ())),
                         preferred_element_type=jnp.float32)


def _tc_root1(x0, w1r, b1):
  BR = 1000
  grid = (N_NODES // BR,)

  def body(x_ref, wr_ref, b_ref, o_ref):
    o_ref[...] = _mm(x_ref[...], wr_ref[...]) + b_ref[...]

  return pl.pallas_call(
      body,
      grid=grid,
      in_specs=[
          pl.BlockSpec((BR, D_IN), lambda i: (i, 0)),
          pl.BlockSpec((D_IN, D_HID), lambda i: (0, 0)),
          pl.BlockSpec((1, D_HID), lambda i: (0, 0)),
      ],
      out_specs=pl.BlockSpec((BR, D_HID), lambda i: (i, 0)),
      out_shape=jax.ShapeDtypeStruct((N_NODES, D_HID), jnp.float32),
  )(x0, w1r, b1)


def _tc_layer1(agg1, r1, w1n):
  BR = 1000
  grid = (N_NODES // BR,)

  def body(agg_ref, r_ref, wn_ref, o_ref):
    h = jnp.maximum(_mm(agg_ref[...], wn_ref[...]) + r_ref[...], 0.0)
    o_ref[0] = h[:, :D_HID // 2]
    o_ref[1] = h[:, D_HID // 2:]

  return pl.pallas_call(
      body,
      grid=grid,
      in_specs=[
          pl.BlockSpec((BR, D_IN), lambda i: (i, 0)),
          pl.BlockSpec((BR, D_HID), lambda i: (i, 0)),
          pl.BlockSpec((D_IN, D_HID), lambda i: (0, 0)),
      ],
      out_specs=pl.BlockSpec((NC, BR, D_HID // 2), lambda i: (0, i, 0)),
      out_shape=jax.ShapeDtypeStruct((NC, N_NODES, D_HID // 2), jnp.float32),
  )(agg1, r1, w1n)


def _tc_root2(x1s, w2r_a, w2r_b, b2, wl_a, wl_b, blin):
  BR = 1000
  grid = (N_NODES // BR,)
  Hh = D_HID // 2

  def body(x1_ref, wra_ref, wrb_ref, b2_ref, wla_ref, wlb_ref, bl_ref,
           o_ref, ol_ref):
    xa = x1_ref[0]
    xb = x1_ref[1]
    o_ref[...] = (_mm(xa, wra_ref[...]) + _mm(xb, wrb_ref[...])
                  + b2_ref[...])
    ol_ref[...] = (_mm(xa, wla_ref[...]) + _mm(xb, wlb_ref[...])
                   + bl_ref[...])

  return pl.pallas_call(
      body,
      grid=grid,
      in_specs=[
          pl.BlockSpec((NC, BR, Hh), lambda i: (0, i, 0)),
          pl.BlockSpec((Hh, D_HID), lambda i: (0, 0)),
          pl.BlockSpec((Hh, D_HID), lambda i: (0, 0)),
          pl.BlockSpec((1, D_HID), lambda i: (0, 0)),
          pl.BlockSpec((Hh, D_OUT), lambda i: (0, 0)),
          pl.BlockSpec((Hh, D_OUT), lambda i: (0, 0)),
          pl.BlockSpec((1, D_OUT), lambda i: (0, 0)),
      ],
      out_specs=[
          pl.BlockSpec((BR, D_HID), lambda i: (i, 0)),
          pl.BlockSpec((BR, D_OUT), lambda i: (i, 0)),
      ],
      out_shape=[
          jax.ShapeDtypeStruct((N_NODES, D_HID), jnp.float32),
          jax.ShapeDtypeStruct((N_NODES, D_OUT), jnp.float32),
      ],
  )(x1s, w2r_a, w2r_b, b2, wl_a, wl_b, blin)


def _tc_layer2(agg2, r2, ylin, w2n_a, w2n_b, wl_2):
  BR = 1000
  grid = (N_NODES // BR,)
  Hh = D_HID // 2

  def body(agg_ref, r_ref, yl_ref, wna_ref, wnb_ref, wl2_ref, o_ref):
    h = _mm(agg_ref[0], wna_ref[...]) + _mm(agg_ref[1], wnb_ref[...])
    x2 = jnp.maximum(h + r_ref[...], 0.0)
    y = yl_ref[...] + _mm(x2, wl2_ref[...])
    m = jnp.max(y, axis=-1, keepdims=True)
    z = y - m
    lse = jnp.log(jnp.sum(jnp.exp(z), axis=-1, keepdims=True))
    o_ref[...] = z - lse

  return pl.pallas_call(
      body,
      grid=grid,
      in_specs=[
          pl.BlockSpec((NC, BR, Hh), lambda i: (0, i, 0)),
          pl.BlockSpec((BR, D_HID), lambda i: (i, 0)),
          pl.BlockSpec((BR, D_OUT), lambda i: (i, 0)),
          pl.BlockSpec((Hh, D_HID), lambda i: (0, 0)),
          pl.BlockSpec((Hh, D_HID), lambda i: (0, 0)),
          pl.BlockSpec((D_HID, D_OUT), lambda i: (0, 0)),
      ],
      out_specs=pl.BlockSpec((BR, D_OUT), lambda i: (i, 0)),
      out_shape=jax.ShapeDtypeStruct((N_NODES, D_OUT), jnp.float32),
  )(agg2, r2, ylin, w2n_a, w2n_b, wl_2)


def kernel(x0, edge_index, edge_weight, W1n, W1r, b1, W2n, W2r, b2,
           Wlin, blin):
  Hh = D_HID // 2

  pad = EPAD - N_EDGES
  zi = jnp.zeros((pad,), jnp.int32)
  src = jnp.concatenate([edge_index[0].astype(jnp.int32), zi])
  src = src.reshape(NS, NCH, CH)
  dst = jnp.concatenate([edge_index[1].astype(jnp.int32), zi])
  dst = dst.reshape(NS, NCH, CH)
  w = jnp.concatenate([edge_weight, jnp.zeros((pad,), jnp.float32)])
  w = w.reshape(NS, NCH, CH)

  agg1 = _sc_seg_l1(x0, src, dst, w)
  agg1 = agg1.reshape(N_NODES, D_IN)
  r1 = _tc_root1(x0, W1r, b1.reshape(1, D_HID))

  x1s = _tc_layer1(agg1, r1, W1n)

  table2 = x1s.reshape(NC * N_NODES, Hh)
  agg2 = _sc_seg_l2(table2, src, dst, w)
  r2, ylin = _tc_root2(x1s, W2r[:Hh], W2r[Hh:], b2.reshape(1, D_HID),
                       Wlin[:Hh], Wlin[Hh:D_HID],
                       blin.reshape(1, D_OUT))

  out = _tc_layer2(agg2, r2, ylin, W2n[:Hh], W2n[Hh:], Wlin[D_HID:])
  return out


# speedup vs baseline: 3.3609x; 3.3609x over previous
import functools

import jax
import jax.numpy as jnp
from jax import lax
from jax.experimental import pallas as pl
from jax.experimental.pallas import tpu as pltpu
from jax.experimental.pallas import tpu_sc as plsc

N_NODES = 10000
N_EDGES = 320000
D_IN = 128
D_HID = 256
D_OUT = 64

NC = 2
NS = 16
CH = 32
NCH = N_EDGES // (NS * CH)
NB = 3
AH = NB - 1
NQ = 12
Dh = 128
G = Dh // 16


def _make_sc_segment_sum(table_rows, acc_rows, dr, zr, mode):
  mesh = plsc.VectorSubcoreMesh(core_axis_name="c", subcore_axis_name="s")

  @functools.partial(
      pl.kernel,
      out_type=jax.ShapeDtypeStruct((NC, acc_rows, Dh), jnp.float32),
      mesh=mesh,
      scratch_types=[
          pltpu.VMEM((NQ, CH), jnp.int32),
          pltpu.VMEM((NQ, CH), jnp.int32),
          pltpu.VMEM((NQ, CH), jnp.float32),
          pltpu.VMEM((NB, CH, Dh), jnp.float32),
          pltpu.VMEM((NB, CH, Dh), jnp.float32),
          pltpu.VMEM((zr, Dh), jnp.float32),
          pltpu.VMEM_SHARED((acc_rows, Dh), jnp.float32),
          pltpu.SemaphoreType.DMA((NB,)),
          pltpu.SemaphoreType.DMA((NB,)),
          pltpu.SemaphoreType.DMA((8,)),
      ],
  )
  def sc_kernel(x_hbm, src_hbm, dst_hbm, w_hbm, out_hbm,
                src_v, dst_v, w_v, rows_v, msg_v, zz_v, acc_sh,
                gsem, ssem, isem):
    c = lax.axis_index("c")
    s = lax.axis_index("s")

    def idx_descs(j):
      q = lax.rem(j, NQ)
      sem = isem.at[lax.rem(j, 8)]
      return ((src_hbm.at[s, j], src_v.at[q], sem),
              (dst_hbm.at[s, j], dst_v.at[q], sem),
              (w_hbm.at[s, j], w_v.at[q], sem))

    def idx_transform(j):
      q = lax.rem(j, NQ)
      if mode == "node":
        lo = c * acc_rows
        for e0 in range(0, CH, 16):
          sl = pl.ds(e0, 16)
          d16 = dst_v[q, sl]
          ok = (d16 >= lo) & (d16 < lo + acc_rows)
          dst_v[q, sl] = jnp.where(ok, d16 - lo, 0)
          w_v[q, sl] = jnp.where(ok, w_v[q, sl], 0.0)
      else:
        off = c * N_NODES
        for e0 in range(0, CH, 16):
          sl = pl.ds(e0, 16)
          src_v[q, sl] = src_v[q, sl] + off

    def idx_start(j):
      for a, v, sem in idx_descs(j):
        pltpu.make_async_copy(a, v, sem).start()

    def idx_wait(j):
      for a, v, sem in idx_descs(j):
        pltpu.make_async_copy(a, v, sem).wait()

    def gather_start(b, k):
      pltpu.make_async_copy(
          x_hbm.at[src_v.at[lax.rem(k, NQ)]], rows_v.at[b],
          gsem.at[b]).start()

    def gather_wait(b, k):
      pltpu.make_async_copy(
          x_hbm.at[src_v.at[lax.rem(k, NQ)]], rows_v.at[b],
          gsem.at[b]).wait()

    def scatter_start(b, k):
      pltpu.make_async_copy(
          msg_v.at[b], acc_sh.at[dst_v.at[lax.rem(k, NQ)]],
          ssem.at[b]).start(add=True)

    def scatter_wait(b, k):
      pltpu.make_async_copy(
          msg_v.at[b], acc_sh.at[dst_v.at[lax.rem(k, NQ)]],
          ssem.at[b]).wait()

    def scale(b, k):
      q = lax.rem(k, NQ)
      for e0 in range(0, CH, 16):
        w16 = w_v[q, pl.ds(e0, 16)]
        for j in range(16):
          ws = jnp.full((16,), w16[j], jnp.float32)
          e = e0 + j
          for g in range(G):
            sl = pl.ds(16 * g, 16)
            msg_v[b, e, sl] = rows_v[b, e, sl] * ws

    for j in range(AH):
      for a, v, _ in idx_descs(j):
        pltpu.sync_copy(a, v)
      idx_transform(jnp.int32(j))
    for j in range(AH, 8):
      idx_start(j)
    for j in range(AH):
      gather_start(j, j)

    zeros16 = jnp.zeros((16,), jnp.float32)

    def zero_row(r, carry):
      for g in range(G):
        zz_v[r, 16 * g:16 * (g + 1)] = zeros16
      return carry

    lax.fori_loop(0, zr, zero_row, 0)
    base = pl.multiple_of(jnp.minimum(dr * s, acc_rows - dr), 8)
    for j in range(dr // zr):
      pltpu.sync_copy(zz_v, acc_sh.at[pl.ds(base + j * zr, zr)])
    plsc.subcore_barrier()

    def step(k, b):
      gather_wait(b, k)
      @pl.when(k >= NB)
      def _wait_prev():
        scatter_wait(b, k - NB)

      @pl.when(k + AH < NCH)
      def _next_gather():
        idx_wait(k + AH)
        idx_transform(k + AH)
        gather_start((b + AH) % NB, k + AH)

      scale(b, k)
      scatter_start(b, k)

      @pl.when(k + 8 < NCH)
      def _next_idx():
        idx_start(k + 8)

    def outer(ko, carry):
      for b in range(NB):
        step(NB * ko + b, b)
      return carry

    lax.fori_loop(0, NCH // NB, outer, 0)

    for k in range(NCH - NCH % NB, NCH):
      step(jnp.int32(k), k % NB)
    for k in range(NCH - NB, NCH):
      scatter_wait(k % NB, k)

    plsc.subcore_barrier()

    pltpu.sync_copy(acc_sh.at[pl.ds(base, dr)],
                    out_hbm.at[c, pl.ds(base, dr)])

  return sc_kernel


_sc_seg_l1 = _make_sc_segment_sum(N_NODES, N_NODES // 2, 320, 80, "node")
_sc_seg_l2 = _make_sc_segment_sum(NC * N_NODES, N_NODES, 640, 128, "col")


def _mm(a, w):
  return lax.dot_general(a, w, (((1,), (0,)), ((), ())),
                         preferred_element_type=jnp.float32)


def _tc_root1(x0, w1r, b1):
  BR = 1000
  grid = (N_NODES // BR,)

  def body(x_ref, wr_ref, b_ref, o_ref):
    o_ref[...] = _mm(x_ref[...], wr_ref[...]) + b_ref[...]

  return pl.pallas_call(
      body,
      grid=grid,
      in_specs=[
          pl.BlockSpec((BR, D_IN), lambda i: (i, 0)),
          pl.BlockSpec((D_IN, D_HID), lambda i: (0, 0)),
          pl.BlockSpec((1, D_HID), lambda i: (0, 0)),
      ],
      out_specs=pl.BlockSpec((BR, D_HID), lambda i: (i, 0)),
      out_shape=jax.ShapeDtypeStruct((N_NODES, D_HID), jnp.float32),
  )(x0, w1r, b1)


def _tc_layer1(agg1, r1, w1n):
  BR = 1000
  grid = (N_NODES // BR,)

  def body(agg_ref, r_ref, wn_ref, o_ref):
    h = jnp.maximum(_mm(agg_ref[...], wn_ref[...]) + r_ref[...], 0.0)
    o_ref[0] = h[:, :D_HID // 2]
    o_ref[1] = h[:, D_HID // 2:]

  return pl.pallas_call(
      body,
      grid=grid,
      in_specs=[
          pl.BlockSpec((BR, D_IN), lambda i: (i, 0)),
          pl.BlockSpec((BR, D_HID), lambda i: (i, 0)),
          pl.BlockSpec((D_IN, D_HID), lambda i: (0, 0)),
      ],
      out_specs=pl.BlockSpec((NC, BR, D_HID // 2), lambda i: (0, i, 0)),
      out_shape=jax.ShapeDtypeStruct((NC, N_NODES, D_HID // 2), jnp.float32),
  )(agg1, r1, w1n)


def _tc_root2(x1s, w2r_a, w2r_b, b2, wl_a, wl_b, blin):
  BR = 1000
  grid = (N_NODES // BR,)
  Hh = D_HID // 2

  def body(x1_ref, wra_ref, wrb_ref, b2_ref, wla_ref, wlb_ref, bl_ref,
           o_ref, ol_ref):
    xa = x1_ref[0]
    xb = x1_ref[1]
    o_ref[...] = (_mm(xa, wra_ref[...]) + _mm(xb, wrb_ref[...])
                  + b2_ref[...])
    ol_ref[...] = (_mm(xa, wla_ref[...]) + _mm(xb, wlb_ref[...])
                   + bl_ref[...])

  return pl.pallas_call(
      body,
      grid=grid,
      in_specs=[
          pl.BlockSpec((NC, BR, Hh), lambda i: (0, i, 0)),
          pl.BlockSpec((Hh, D_HID), lambda i: (0, 0)),
          pl.BlockSpec((Hh, D_HID), lambda i: (0, 0)),
          pl.BlockSpec((1, D_HID), lambda i: (0, 0)),
          pl.BlockSpec((Hh, D_OUT), lambda i: (0, 0)),
          pl.BlockSpec((Hh, D_OUT), lambda i: (0, 0)),
          pl.BlockSpec((1, D_OUT), lambda i: (0, 0)),
      ],
      out_specs=[
          pl.BlockSpec((BR, D_HID), lambda i: (i, 0)),
          pl.BlockSpec((BR, D_OUT), lambda i: (i, 0)),
      ],
      out_shape=[
          jax.ShapeDtypeStruct((N_NODES, D_HID), jnp.float32),
          jax.ShapeDtypeStruct((N_NODES, D_OUT), jnp.float32),
      ],
  )(x1s, w2r_a, w2r_b, b2, wl_a, wl_b, blin)


def _tc_layer2(agg2, r2, ylin, w2n_a, w2n_b, wl_2):
  BR = 1000
  grid = (N_NODES // BR,)
  Hh = D_HID // 2

  def body(agg_ref, r_ref, yl_ref, wna_ref, wnb_ref, wl2_ref, o_ref):
    h = _mm(agg_ref[0], wna_ref[...]) + _mm(agg_ref[1], wnb_ref[...])
    x2 = jnp.maximum(h + r_ref[...], 0.0)
    y = yl_ref[...] + _mm(x2, wl2_ref[...])
    m = jnp.max(y, axis=-1, keepdims=True)
    z = y - m
    lse = jnp.log(jnp.sum(jnp.exp(z), axis=-1, keepdims=True))
    o_ref[...] = z - lse

  return pl.pallas_call(
      body,
      grid=grid,
      in_specs=[
          pl.BlockSpec((NC, BR, Hh), lambda i: (0, i, 0)),
          pl.BlockSpec((BR, D_HID), lambda i: (i, 0)),
          pl.BlockSpec((BR, D_OUT), lambda i: (i, 0)),
          pl.BlockSpec((Hh, D_HID), lambda i: (0, 0)),
          pl.BlockSpec((Hh, D_HID), lambda i: (0, 0)),
          pl.BlockSpec((D_HID, D_OUT), lambda i: (0, 0)),
      ],
      out_specs=pl.BlockSpec((BR, D_OUT), lambda i: (i, 0)),
      out_shape=jax.ShapeDtypeStruct((N_NODES, D_OUT), jnp.float32),
  )(agg2, r2, ylin, w2n_a, w2n_b, wl_2)


def kernel(x0, edge_index, edge_weight, W1n, W1r, b1, W2n, W2r, b2,
           Wlin, blin):
  Hh = D_HID // 2

  src = edge_index[0].astype(jnp.int32).reshape(NS, NCH, CH)
  dst = edge_index[1].astype(jnp.int32).reshape(NS, NCH, CH)
  w = edge_weight.reshape(NS, NCH, CH)

  agg1 = _sc_seg_l1(x0, src, dst, w)
  agg1 = agg1.reshape(N_NODES, D_IN)
  r1 = _tc_root1(x0, W1r, b1.reshape(1, D_HID))

  x1s = _tc_layer1(agg1, r1, W1n)

  table2 = x1s.reshape(NC * N_NODES, Hh)
  agg2 = _sc_seg_l2(table2, src, dst, w)
  r2, ylin = _tc_root2(x1s, W2r[:Hh], W2r[Hh:], b2.reshape(1, D_HID),
                       Wlin[:Hh], Wlin[Hh:D_HID],
                       blin.reshape(1, D_OUT))

  out = _tc_layer2(agg2, r2, ylin, W2n[:Hh], W2n[Hh:], Wlin[D_HID:])
  return out


# speedup vs baseline: 3.3709x; 1.0030x over previous
import functools

import jax
import jax.numpy as jnp
from jax import lax
from jax.experimental import pallas as pl
from jax.experimental.pallas import tpu as pltpu
from jax.experimental.pallas import tpu_sc as plsc

N_NODES = 10000
N_EDGES = 320000
D_IN = 128
D_HID = 256
D_OUT = 64

NC = 2
NS = 16
CH = 32
NCH = N_EDGES // (NS * CH)
NB = 3
AH = NB - 1
NQ = 12
Dh = 128
G = Dh // 16


def _make_sc_segment_sum(table_rows, acc_rows, dr, zr, mode):
  mesh = plsc.VectorSubcoreMesh(core_axis_name="c", subcore_axis_name="s")

  @functools.partial(
      pl.kernel,
      out_type=jax.ShapeDtypeStruct((NC, acc_rows, Dh), jnp.float32),
      mesh=mesh,
      scratch_types=[
          pltpu.VMEM((NQ, CH), jnp.int32),
          pltpu.VMEM((NQ, CH), jnp.int32),
          pltpu.VMEM((NQ, CH), jnp.float32),
          pltpu.VMEM((NB, CH, Dh), jnp.float32),
          pltpu.VMEM((NB, CH, Dh), jnp.float32),
          pltpu.VMEM((zr, Dh), jnp.float32),
          pltpu.VMEM_SHARED((acc_rows, Dh), jnp.float32),
          pltpu.SemaphoreType.DMA((NB,)),
          pltpu.SemaphoreType.DMA((NB,)),
          pltpu.SemaphoreType.DMA((8,)),
      ],
  )
  def sc_kernel(x_hbm, src_hbm, dst_hbm, w_hbm, out_hbm,
                src_v, dst_v, w_v, rows_v, msg_v, zz_v, acc_sh,
                gsem, ssem, isem):
    c = lax.axis_index("c")
    s = lax.axis_index("s")

    def idx_descs(j):
      q = lax.rem(j, NQ)
      sem = isem.at[lax.rem(j, 8)]
      return ((src_hbm.at[s, j], src_v.at[q], sem),
              (dst_hbm.at[s, j], dst_v.at[q], sem),
              (w_hbm.at[s, j], w_v.at[q], sem))

    def idx_transform(j):
      q = lax.rem(j, NQ)
      if mode == "node":
        lo = c * acc_rows
        for e0 in range(0, CH, 16):
          sl = pl.ds(e0, 16)
          d16 = dst_v[q, sl]
          ok = (d16 >= lo) & (d16 < lo + acc_rows)
          dst_v[q, sl] = jnp.where(ok, d16 - lo, 0)
          w_v[q, sl] = jnp.where(ok, w_v[q, sl], 0.0)
      else:
        off = c * N_NODES
        for e0 in range(0, CH, 16):
          sl = pl.ds(e0, 16)
          src_v[q, sl] = src_v[q, sl] + off

    def idx_start(j):
      for a, v, sem in idx_descs(j):
        pltpu.make_async_copy(a, v, sem).start()

    def idx_wait(j):
      for a, v, sem in idx_descs(j):
        pltpu.make_async_copy(a, v, sem).wait()

    def gather_start(b, k):
      pltpu.make_async_copy(
          x_hbm.at[src_v.at[lax.rem(k, NQ)]], rows_v.at[b],
          gsem.at[b]).start()

    def gather_wait(b, k):
      pltpu.make_async_copy(
          x_hbm.at[src_v.at[lax.rem(k, NQ)]], rows_v.at[b],
          gsem.at[b]).wait()

    def scatter_start(b, k):
      pltpu.make_async_copy(
          msg_v.at[b], acc_sh.at[dst_v.at[lax.rem(k, NQ)]],
          ssem.at[b]).start(add=True)

    def scatter_wait(b, k):
      pltpu.make_async_copy(
          msg_v.at[b], acc_sh.at[dst_v.at[lax.rem(k, NQ)]],
          ssem.at[b]).wait()

    def scale(b, k):
      q = lax.rem(k, NQ)
      for e0 in range(0, CH, 16):
        w16 = w_v[q, pl.ds(e0, 16)]
        for j in range(16):
          ws = jnp.full((16,), w16[j], jnp.float32)
          e = e0 + j
          for g in range(G):
            sl = pl.ds(16 * g, 16)
            msg_v[b, e, sl] = rows_v[b, e, sl] * ws

    for j in range(AH):
      for a, v, _ in idx_descs(j):
        pltpu.sync_copy(a, v)
      idx_transform(jnp.int32(j))
    for j in range(AH, 8):
      idx_start(j)
    for j in range(AH):
      gather_start(j, j)

    zeros16 = jnp.zeros((16,), jnp.float32)

    def zero_row(r, carry):
      for g in range(G):
        zz_v[r, 16 * g:16 * (g + 1)] = zeros16
      return carry

    lax.fori_loop(0, zr, zero_row, 0)
    base = pl.multiple_of(jnp.minimum(dr * s, acc_rows - dr), 8)
    for j in range(dr // zr):
      pltpu.sync_copy(zz_v, acc_sh.at[pl.ds(base + j * zr, zr)])
    plsc.subcore_barrier()

    def step(k, b):
      gather_wait(b, k)
      @pl.when(k >= NB)
      def _wait_prev():
        scatter_wait(b, k - NB)

      @pl.when(k + AH < NCH)
      def _next_gather():
        idx_wait(k + AH)
        idx_transform(k + AH)
        gather_start((b + AH) % NB, k + AH)

      scale(b, k)
      scatter_start(b, k)

      @pl.when(k + 8 < NCH)
      def _next_idx():
        idx_start(k + 8)

    def outer(ko, carry):
      for b in range(NB):
        step(NB * ko + b, b)
      return carry

    lax.fori_loop(0, NCH // NB, outer, 0)

    for k in range(NCH - NCH % NB, NCH):
      step(jnp.int32(k), k % NB)
    for k in range(NCH - NB, NCH):
      scatter_wait(k % NB, k)

    plsc.subcore_barrier()

    pltpu.sync_copy(acc_sh.at[pl.ds(base, dr)],
                    out_hbm.at[c, pl.ds(base, dr)])

  return sc_kernel


_sc_seg_l1 = _make_sc_segment_sum(N_NODES, N_NODES // 2, 320, 80, "node")
_sc_seg_l2 = _make_sc_segment_sum(NC * N_NODES, N_NODES, 640, 128, "col")


def _mm(a, w):
  return lax.dot_general(a, w, (((1,), (0,)), ((), ())),
                         preferred_element_type=jnp.float32)


def _tc_layer1(agg1, x0, w1n, w1r, b1):
  BR = 1000
  grid = (N_NODES // BR,)

  def body(agg_ref, x_ref, wn_ref, wr_ref, b_ref, o_ref):
    h = _mm(agg_ref[...], wn_ref[...]) + _mm(x_ref[...], wr_ref[...])
    h = jnp.maximum(h + b_ref[...], 0.0)
    o_ref[0] = h[:, :D_HID // 2]
    o_ref[1] = h[:, D_HID // 2:]

  return pl.pallas_call(
      body,
      grid=grid,
      in_specs=[
          pl.BlockSpec((BR, D_IN), lambda i: (i, 0)),
          pl.BlockSpec((BR, D_IN), lambda i: (i, 0)),
          pl.BlockSpec((D_IN, D_HID), lambda i: (0, 0)),
          pl.BlockSpec((D_IN, D_HID), lambda i: (0, 0)),
          pl.BlockSpec((1, D_HID), lambda i: (0, 0)),
      ],
      out_specs=pl.BlockSpec((NC, BR, D_HID // 2), lambda i: (0, i, 0)),
      out_shape=jax.ShapeDtypeStruct((NC, N_NODES, D_HID // 2), jnp.float32),
  )(agg1, x0, w1n, w1r, b1)


def _tc_layer2(agg2, x1s, w2n_a, w2n_b, w2r_a, w2r_b, b2,
               wl_a, wl_b, wl_2, blin):
  BR = 1000
  grid = (N_NODES // BR,)
  Hh = D_HID // 2

  def body(agg_ref, x1_ref, wna_ref, wnb_ref, wra_ref, wrb_ref, b2_ref,
           wla_ref, wlb_ref, wl2_ref, bl_ref, o_ref):
    xa = x1_ref[0]
    xb = x1_ref[1]
    h = _mm(agg_ref[0], wna_ref[...]) + _mm(agg_ref[1], wnb_ref[...])
    h += _mm(xa, wra_ref[...]) + _mm(xb, wrb_ref[...])
    x2 = jnp.maximum(h + b2_ref[...], 0.0)
    y = _mm(xa, wla_ref[...]) + _mm(xb, wlb_ref[...]) + _mm(x2, wl2_ref[...])
    y += bl_ref[...]
    m = jnp.max(y, axis=-1, keepdims=True)
    z = y - m
    lse = jnp.log(jnp.sum(jnp.exp(z), axis=-1, keepdims=True))
    o_ref[...] = z - lse

  return pl.pallas_call(
      body,
      grid=grid,
      in_specs=[
          pl.BlockSpec((NC, BR, Hh), lambda i: (0, i, 0)),
          pl.BlockSpec((NC, BR, Hh), lambda i: (0, i, 0)),
          pl.BlockSpec((Hh, D_HID), lambda i: (0, 0)),
          pl.BlockSpec((Hh, D_HID), lambda i: (0, 0)),
          pl.BlockSpec((Hh, D_HID), lambda i: (0, 0)),
          pl.BlockSpec((Hh, D_HID), lambda i: (0, 0)),
          pl.BlockSpec((1, D_HID), lambda i: (0, 0)),
          pl.BlockSpec((Hh, D_OUT), lambda i: (0, 0)),
          pl.BlockSpec((Hh, D_OUT), lambda i: (0, 0)),
          pl.BlockSpec((D_HID, D_OUT), lambda i: (0, 0)),
          pl.BlockSpec((1, D_OUT), lambda i: (0, 0)),
      ],
      out_specs=pl.BlockSpec((BR, D_OUT), lambda i: (i, 0)),
      out_shape=jax.ShapeDtypeStruct((N_NODES, D_OUT), jnp.float32),
  )(agg2, x1s, w2n_a, w2n_b, w2r_a, w2r_b, b2, wl_a, wl_b, wl_2, blin)


def kernel(x0, edge_index, edge_weight, W1n, W1r, b1, W2n, W2r, b2,
           Wlin, blin):
  Hh = D_HID // 2

  src = edge_index[0].astype(jnp.int32).reshape(NS, NCH, CH)
  dst = edge_index[1].astype(jnp.int32).reshape(NS, NCH, CH)
  w = edge_weight.reshape(NS, NCH, CH)

  agg1 = _sc_seg_l1(x0, src, dst, w)
  agg1 = agg1.reshape(N_NODES, D_IN)

  x1s = _tc_layer1(agg1, x0, W1n, W1r, b1.reshape(1, D_HID))

  table2 = x1s.reshape(NC * N_NODES, Hh)
  agg2 = _sc_seg_l2(table2, src, dst, w)

  out = _tc_layer2(agg2, x1s, W2n[:Hh], W2n[Hh:], W2r[:Hh], W2r[Hh:],
                   b2.reshape(1, D_HID), Wlin[:Hh], Wlin[Hh:D_HID],
                   Wlin[D_HID:], blin.reshape(1, D_OUT))
  return out
